# baseline (speedup 1.0000x reference)
import functools

import jax
import jax.numpy as jnp
from jax import lax
from jax.experimental import pallas as pl
from jax.experimental.pallas import tpu as pltpu
from jax.experimental.pallas import tpu_sc as plsc

_N = 10000
_E = 320000
_D = 128
_H = 128
_C = 64

_NC = 2
_NS = 16
_NW = _NC * _NS
_K = 80
_EP = _E // _NW
_NCHUNK = _EP // _K
_NR = _N
_KZ = 80
_NZ = _N // _KZ
_ZPT = -(-_NZ // _NS)


def _make_sc_agg(df, with_deg, nb, nib):
  unroll = max(nb, nib)
  nout = -(-_NCHUNK // unroll)
  mesh = plsc.VectorSubcoreMesh(core_axis_name="c", subcore_axis_name="s")
  out_type = [jax.ShapeDtypeStruct((_NC, _N, df), jnp.float32)]
  scratch = [
      [pltpu.VMEM((2, _K), jnp.int32) for _ in range(nib)],
      [pltpu.VMEM((_K, df), jnp.float32) for _ in range(nb)],
      pltpu.VMEM_SHARED((_NR, df), jnp.float32),
      [pltpu.SemaphoreType.DMA for _ in range(nib)],
      [pltpu.SemaphoreType.DMA for _ in range(nb)],
  ]
  if with_deg:
    out_type.append(jax.ShapeDtypeStruct((_NC * _N,), jnp.float32))
    scratch += [
        pltpu.VMEM((_K,), jnp.float32),
        pltpu.VMEM((_KZ,), jnp.float32),
        pltpu.VMEM_SHARED((_NR,), jnp.float32),
    ]

  def body(feat_hbm, sidx_hbm, agg_out, *rest):
    if with_deg:
      deg_out, idx_v, rows_v, agg_sh, isems, gsems, ones_v, zeros_v, \
          deg_sh = rest
    else:
      idx_v, rows_v, agg_sh, isems, gsems = rest
      deg_out = ones_v = zeros_v = deg_sh = None

    c = lax.axis_index("c")
    s = lax.axis_index("s")
    wid = c * _NS + s

    for b in range(nib - 2):
      pltpu.async_copy(sidx_hbm.at[wid, b], idx_v[b], isems[b])

    zvec = jnp.zeros((16,), jnp.float32)

    def zrow(r, carry):
      for g in range(df // 16):
        rows_v[0][r, pl.ds(g * 16, 16)] = zvec
      return carry
    lax.fori_loop(0, _KZ, zrow, 0)
    if with_deg:
      for g in range(_K // 16):
        ones_v[pl.ds(g * 16, 16)] = jnp.ones((16,), jnp.float32)
      for g in range(_KZ // 16):
        zeros_v[pl.ds(g * 16, 16)] = zvec

    for jj in range(_ZPT):
      j = s * _ZPT + jj

      @pl.when(j < _NZ)
      def _():
        pltpu.sync_copy(rows_v[0], agg_sh.at[pl.ds(j * _KZ, _KZ)])
        if with_deg:
          pltpu.sync_copy(zeros_v, deg_sh.at[pl.ds(j * _KZ, _KZ)])

    plsc.subcore_barrier()

    for b in range(nb - 1):
      pltpu.make_async_copy(sidx_hbm.at[wid, b], idx_v[b], isems[b]).wait()
      pltpu.async_copy(feat_hbm.at[idx_v[b].at[0]], rows_v[b], gsems[b])

    def ebody(i, carry):
      for b in range(unroll):
        j = i * unroll + b

        @pl.when(j < _NCHUNK)
        def _():
          jg = j + nb - 1
          bg = (b + nb - 1) % nb
          ig = (b + nb - 1) % nib

          @pl.when(jg < _NCHUNK)
          def _():
            pltpu.make_async_copy(sidx_hbm.at[wid, jg], idx_v[ig],
                                  isems[ig]).wait()
            pltpu.async_copy(feat_hbm.at[idx_v[ig].at[0]], rows_v[bg],
                             gsems[bg])

          pltpu.make_async_copy(feat_hbm.at[idx_v[b % nib].at[0]],
                                rows_v[b % nb], gsems[b % nb]).wait()
          pltpu.sync_copy(rows_v[b % nb], agg_sh.at[idx_v[b % nib].at[1]],
                          add=True)
          if with_deg:
            pltpu.sync_copy(ones_v, deg_sh.at[idx_v[b % nib].at[1]],
                            add=True)

          ji = j + nib - 2
          bi = (b + nib - 2) % nib

          @pl.when(ji < _NCHUNK)
          def _():
            pltpu.async_copy(sidx_hbm.at[wid, ji], idx_v[bi], isems[bi])
      return carry
    lax.fori_loop(0, nout, ebody, 0)

    plsc.subcore_barrier()

    for jj in range(_ZPT):
      j = s * _ZPT + jj

      @pl.when(j < _NZ)
      def _():
        pltpu.sync_copy(agg_sh.at[pl.ds(j * _KZ, _KZ)],
                        agg_out.at[c, pl.ds(j * _KZ, _KZ)])
        if with_deg:
          pltpu.sync_copy(deg_sh.at[pl.ds(j * _KZ, _KZ)], zeros_v)
          pltpu.sync_copy(zeros_v, deg_out.at[pl.ds(c * _N + j * _KZ, _KZ)])

  return pl.kernel(
      body, out_type=out_type, mesh=mesh, scratch_types=scratch,
      compiler_params=pltpu.CompilerParams(
          use_tc_tiling_on_sc=False, disable_bounds_checks=True,
          disable_semaphore_checks=True))


_sc_agg_deg = _make_sc_agg(_D, True, 4, 8)
_sc_agg = _make_sc_agg(_C, False, 8, 16)

_BM = 1000


def _tc1_body(agg_ref, degt_ref, x_ref, w1l_ref, w1r_ref, b1_ref,
              w2l_ref, w2r_ref, b2_ref, p_ref, q_ref):
  agg = agg_ref[0] + agg_ref[1]
  degt = degt_ref[...]
  deg = jnp.maximum(degt[:, 0] + degt[:, 1], 1.0)
  mean = agg / deg[:, None]
  h = mean @ w1l_ref[...] + x_ref[...] @ w1r_ref[...] + b1_ref[...]
  h = jnp.maximum(h, 0.0)
  p_ref[...] = h @ w2l_ref[...]
  q_ref[...] = h @ w2r_ref[...] + b2_ref[...]


def _tc2_body(agg_ref, degt_ref, q_ref, out_ref):
  agg = agg_ref[0] + agg_ref[1]
  degt = degt_ref[...]
  deg = jnp.maximum(degt[:, 0] + degt[:, 1], 1.0)
  out_ref[...] = agg / deg[:, None] + q_ref[...]


_tc1 = pl.pallas_call(
    _tc1_body,
    grid=(_N // _BM,),
    in_specs=[
        pl.BlockSpec((_NC, _BM, _D), lambda i: (0, i, 0)),
        pl.BlockSpec((_BM, _NC), lambda i: (i, 0)),
        pl.BlockSpec((_BM, _D), lambda i: (i, 0)),
        pl.BlockSpec((_D, _H), lambda i: (0, 0)),
        pl.BlockSpec((_D, _H), lambda i: (0, 0)),
        pl.BlockSpec((1, _H), lambda i: (0, 0)),
        pl.BlockSpec((_H, _C), lambda i: (0, 0)),
        pl.BlockSpec((_H, _C), lambda i: (0, 0)),
        pl.BlockSpec((1, _C), lambda i: (0, 0)),
    ],
    out_specs=[
        pl.BlockSpec((_BM, _C), lambda i: (i, 0)),
        pl.BlockSpec((_BM, _C), lambda i: (i, 0)),
    ],
    out_shape=[
        jax.ShapeDtypeStruct((_N, _C), jnp.float32),
        jax.ShapeDtypeStruct((_N, _C), jnp.float32),
    ],
)

_tc2 = pl.pallas_call(
    _tc2_body,
    grid=(_N // _BM,),
    in_specs=[
        pl.BlockSpec((_NC, _BM, _C), lambda i: (0, i, 0)),
        pl.BlockSpec((_BM, _NC), lambda i: (i, 0)),
        pl.BlockSpec((_BM, _C), lambda i: (i, 0)),
    ],
    out_specs=pl.BlockSpec((_BM, _C), lambda i: (i, 0)),
    out_shape=jax.ShapeDtypeStruct((_N, _C), jnp.float32),
)


@jax.jit
def kernel(x, edge_index, W1l, W1r, b1, W2l, W2r, b2):
  ei = edge_index.astype(jnp.int32)
  sidx = jnp.stack([ei[0].reshape(_NW, _NCHUNK, _K),
                    ei[1].reshape(_NW, _NCHUNK, _K)], axis=2)

  agg1, deg = _sc_agg_deg(x, sidx)
  degt = deg.reshape(_NC, _N).T

  p, q = _tc1(agg1, degt, x, W1l, W1r, b1.reshape(1, _H),
              W2l, W2r, b2.reshape(1, _C))

  agg2, = _sc_agg(p, sidx)
  out = _tc2(agg2, degt, q)
  return out


# speedup vs baseline: 1.0023x; 1.0023x over previous
import functools

import jax
import jax.numpy as jnp
from jax import lax
from jax.experimental import pallas as pl
from jax.experimental.pallas import tpu as pltpu
from jax.experimental.pallas import tpu_sc as plsc

_N = 10000
_E = 320000
_D = 128
_H = 128
_C = 64

_NC = 2
_NS = 16
_NW = _NC * _NS
_K = 80
_EP = _E // _NW
_NCHUNK = _EP // _K
_NR = _N
_KZ = 80
_NZ = _N // _KZ
_ZPT = -(-_NZ // _NS)


def _make_sc_agg(df, with_deg, nb, nib):
  unroll = max(nb, nib)
  nout = -(-_NCHUNK // unroll)
  mesh = plsc.VectorSubcoreMesh(core_axis_name="c", subcore_axis_name="s")
  out_type = [jax.ShapeDtypeStruct((_NC, _N, df), jnp.float32)]
  scratch = [
      [pltpu.VMEM((2, _K), jnp.int32) for _ in range(nib)],
      [pltpu.VMEM((_K, df), jnp.float32) for _ in range(nb)],
      pltpu.VMEM_SHARED((_NR, df), jnp.float32),
      [pltpu.SemaphoreType.DMA for _ in range(nib)],
      [pltpu.SemaphoreType.DMA for _ in range(nb)],
  ]
  if with_deg:
    out_type.append(jax.ShapeDtypeStruct((_NC * _N,), jnp.float32))
    scratch += [
        pltpu.VMEM((_K,), jnp.float32),
        pltpu.VMEM((_KZ,), jnp.float32),
        pltpu.VMEM_SHARED((_NR,), jnp.float32),
        [pltpu.SemaphoreType.DMA for _ in range(2)],
    ]

  def body(feat_hbm, sidx_hbm, agg_out, *rest):
    if with_deg:
      deg_out, idx_v, rows_v, agg_sh, isems, gsems, ones_v, zeros_v, \
          deg_sh, dsems = rest
    else:
      idx_v, rows_v, agg_sh, isems, gsems = rest
      deg_out = ones_v = zeros_v = deg_sh = dsems = None

    c = lax.axis_index("c")
    s = lax.axis_index("s")
    wid = c * _NS + s

    for b in range(nib - 2):
      pltpu.async_copy(sidx_hbm.at[wid, b], idx_v[b], isems[b])

    zvec = jnp.zeros((16,), jnp.float32)

    def zrow(r, carry):
      for g in range(df // 16):
        rows_v[0][r, pl.ds(g * 16, 16)] = zvec
      return carry
    lax.fori_loop(0, _KZ, zrow, 0)
    if with_deg:
      for g in range(_K // 16):
        ones_v[pl.ds(g * 16, 16)] = jnp.ones((16,), jnp.float32)
      for g in range(_KZ // 16):
        zeros_v[pl.ds(g * 16, 16)] = zvec

    for jj in range(_ZPT):
      j = s * _ZPT + jj

      @pl.when(j < _NZ)
      def _():
        pltpu.sync_copy(rows_v[0], agg_sh.at[pl.ds(j * _KZ, _KZ)])
        if with_deg:
          pltpu.sync_copy(zeros_v, deg_sh.at[pl.ds(j * _KZ, _KZ)])

    plsc.subcore_barrier()

    for b in range(nb - 1):
      pltpu.make_async_copy(sidx_hbm.at[wid, b], idx_v[b], isems[b]).wait()
      pltpu.async_copy(feat_hbm.at[idx_v[b].at[0]], rows_v[b], gsems[b])

    def ebody(i, carry):
      for b in range(unroll):
        j = i * unroll + b

        @pl.when(j < _NCHUNK)
        def _():
          jg = j + nb - 1
          bg = (b + nb - 1) % nb
          ig = (b + nb - 1) % nib

          @pl.when(jg < _NCHUNK)
          def _():
            pltpu.make_async_copy(sidx_hbm.at[wid, jg], idx_v[ig],
                                  isems[ig]).wait()
            pltpu.async_copy(feat_hbm.at[idx_v[ig].at[0]], rows_v[bg],
                             gsems[bg])

          pltpu.make_async_copy(feat_hbm.at[idx_v[b % nib].at[0]],
                                rows_v[b % nb], gsems[b % nb]).wait()
          pltpu.sync_copy(rows_v[b % nb], agg_sh.at[idx_v[b % nib].at[1]],
                          add=True)
          if with_deg:
            @pl.when(j >= 1)
            def _():
              pltpu.make_async_copy(
                  ones_v, deg_sh.at[idx_v[(b - 1) % nib].at[1]],
                  dsems[(b - 1) % 2]).wait()
            pltpu.async_copy(ones_v, deg_sh.at[idx_v[b % nib].at[1]],
                             dsems[b % 2], add=True)

          ji = j + nib - 2
          bi = (b + nib - 2) % nib

          @pl.when(ji < _NCHUNK)
          def _():
            pltpu.async_copy(sidx_hbm.at[wid, ji], idx_v[bi], isems[bi])
      return carry
    lax.fori_loop(0, nout, ebody, 0)

    if with_deg:
      pltpu.make_async_copy(
          ones_v, deg_sh.at[idx_v[(_NCHUNK - 1) % nib].at[1]],
          dsems[(_NCHUNK - 1) % 2]).wait()

    plsc.subcore_barrier()

    for jj in range(_ZPT):
      j = s * _ZPT + jj

      @pl.when(j < _NZ)
      def _():
        pltpu.sync_copy(agg_sh.at[pl.ds(j * _KZ, _KZ)],
                        agg_out.at[c, pl.ds(j * _KZ, _KZ)])
        if with_deg:
          pltpu.sync_copy(deg_sh.at[pl.ds(j * _KZ, _KZ)], zeros_v)
          pltpu.sync_copy(zeros_v, deg_out.at[pl.ds(c * _N + j * _KZ, _KZ)])

  return pl.kernel(
      body, out_type=out_type, mesh=mesh, scratch_types=scratch,
      compiler_params=pltpu.CompilerParams(
          use_tc_tiling_on_sc=False, disable_bounds_checks=True,
          disable_semaphore_checks=True, skip_device_barrier=True))


_sc_agg_deg = _make_sc_agg(_D, True, 4, 8)
_sc_agg = _make_sc_agg(_C, False, 8, 16)

_BM = 1000


def _tc1_body(agg_ref, degt_ref, x_ref, w1l_ref, w1r_ref, b1_ref,
              w2l_ref, w2r_ref, b2_ref, p_ref, q_ref):
  agg = agg_ref[0] + agg_ref[1]
  degt = degt_ref[...]
  deg = jnp.maximum(degt[:, 0] + degt[:, 1], 1.0)
  mean = agg / deg[:, None]
  h = mean @ w1l_ref[...] + x_ref[...] @ w1r_ref[...] + b1_ref[...]
  h = jnp.maximum(h, 0.0)
  p_ref[...] = h @ w2l_ref[...]
  q_ref[...] = h @ w2r_ref[...] + b2_ref[...]


def _tc2_body(agg_ref, degt_ref, q_ref, out_ref):
  agg = agg_ref[0] + agg_ref[1]
  degt = degt_ref[...]
  deg = jnp.maximum(degt[:, 0] + degt[:, 1], 1.0)
  out_ref[...] = agg / deg[:, None] + q_ref[...]


_tc1 = pl.pallas_call(
    _tc1_body,
    grid=(_N // _BM,),
    in_specs=[
        pl.BlockSpec((_NC, _BM, _D), lambda i: (0, i, 0)),
        pl.BlockSpec((_BM, _NC), lambda i: (i, 0)),
        pl.BlockSpec((_BM, _D), lambda i: (i, 0)),
        pl.BlockSpec((_D, _H), lambda i: (0, 0)),
        pl.BlockSpec((_D, _H), lambda i: (0, 0)),
        pl.BlockSpec((1, _H), lambda i: (0, 0)),
        pl.BlockSpec((_H, _C), lambda i: (0, 0)),
        pl.BlockSpec((_H, _C), lambda i: (0, 0)),
        pl.BlockSpec((1, _C), lambda i: (0, 0)),
    ],
    out_specs=[
        pl.BlockSpec((_BM, _C), lambda i: (i, 0)),
        pl.BlockSpec((_BM, _C), lambda i: (i, 0)),
    ],
    out_shape=[
        jax.ShapeDtypeStruct((_N, _C), jnp.float32),
        jax.ShapeDtypeStruct((_N, _C), jnp.float32),
    ],
)

_tc2 = pl.pallas_call(
    _tc2_body,
    grid=(_N // _BM,),
    in_specs=[
        pl.BlockSpec((_NC, _BM, _C), lambda i: (0, i, 0)),
        pl.BlockSpec((_BM, _NC), lambda i: (i, 0)),
        pl.BlockSpec((_BM, _C), lambda i: (i, 0)),
    ],
    out_specs=pl.BlockSpec((_BM, _C), lambda i: (i, 0)),
    out_shape=jax.ShapeDtypeStruct((_N, _C), jnp.float32),
)


@jax.jit
def kernel(x, edge_index, W1l, W1r, b1, W2l, W2r, b2):
  ei = edge_index.astype(jnp.int32)
  sidx = jnp.stack([ei[0].reshape(_NW, _NCHUNK, _K),
                    ei[1].reshape(_NW, _NCHUNK, _K)], axis=2)

  agg1, deg = _sc_agg_deg(x, sidx)
  degt = deg.reshape(_NC, _N).T

  p, q = _tc1(agg1, degt, x, W1l, W1r, b1.reshape(1, _H),
              W2l, W2r, b2.reshape(1, _C))

  agg2, = _sc_agg(p, sidx)
  out = _tc2(agg2, degt, q)
  return out
